# Initial kernel scaffold; baseline (speedup 1.0000x reference)
import jax
import jax.numpy as jnp
from jax.experimental import pallas as pl

N_PAIR = 32
N_CTX = 8
MAX_L = 40
D = 768
NUM_BASE = 92
NUM_NOVEL = 40
N_CLS = NUM_BASE + NUM_NOVEL
VOCAB = 1000
D_ENTI = 256
SUF_L = MAX_L - 1 - N_CTX

C_BLK = 22
NCC = N_CLS // C_BLK


def _ctx_body(ids_ref, enti_ref, sW1_ref, sb1_ref, sW2_ref, oW1_ref, ob1_ref,
              oW2_ref, sctx_in_ref, octx_in_ref, tm_ref,
              sctx_ref, octx_ref, tm3_ref):
    ids = ids_ref[:]
    iota = jax.lax.broadcasted_iota(jnp.int32, (N_PAIR, VOCAB), 1)
    s_oh = (ids[:, 0:1] == iota).astype(jnp.float32)
    o_oh = (ids[:, 1:2] == iota).astype(jnp.float32)
    enti = enti_ref[:]
    s_e = jnp.dot(s_oh, enti, preferred_element_type=jnp.float32)
    o_e = jnp.dot(o_oh, enti, preferred_element_type=jnp.float32)
    s_h = jnp.maximum(
        jnp.dot(s_e, sW1_ref[:], preferred_element_type=jnp.float32) + sb1_ref[:], 0.0)
    o_h = jnp.maximum(
        jnp.dot(o_e, oW1_ref[:], preferred_element_type=jnp.float32) + ob1_ref[:], 0.0)
    s_emb = jnp.dot(s_h, sW2_ref[:], preferred_element_type=jnp.float32)
    o_emb = jnp.dot(o_h, oW2_ref[:], preferred_element_type=jnp.float32)
    sctx_ref[:] = sctx_in_ref[:][None, :, :] + s_emb[:, None, :]
    octx_ref[:] = octx_in_ref[:][None, :, :] + o_emb[:, None, :]
    tm3_ref[:] = jnp.broadcast_to(tm_ref[:][None, :, :], (N_PAIR, N_CLS, MAX_L))


def _assemble_body(pre_ref, suf_ref, sctx_ref, octx_ref, subj_ref, obj_ref):
    pre = pre_ref[:][:, None, :]
    suf = suf_ref[:]
    s_ctx = jnp.broadcast_to(sctx_ref[0][None, :, :], (C_BLK, N_CTX, D))
    o_ctx = jnp.broadcast_to(octx_ref[0][None, :, :], (C_BLK, N_CTX, D))
    subj_ref[:] = jnp.concatenate([pre, s_ctx, suf], axis=1)
    obj_ref[:] = jnp.concatenate([pre, o_ctx, suf], axis=1)


def kernel(so_cls_ids, enti_txt_embds, prefix_embds, suffix_embds, token_mask,
           subj_ctx_embds, obj_ctx_embds, sW1, sb1, sW2, oW1, ob1, oW2):
    prefix_sl = prefix_embds[1:N_CLS + 1, 0, :]
    suffix_sl = suffix_embds[1:N_CLS + 1]
    tm_sl = token_mask[1:N_CLS + 1]

    s_ctx, o_ctx, tm3 = pl.pallas_call(
        _ctx_body,
        out_shape=(
            jax.ShapeDtypeStruct((N_PAIR, N_CTX, D), jnp.float32),
            jax.ShapeDtypeStruct((N_PAIR, N_CTX, D), jnp.float32),
            jax.ShapeDtypeStruct((N_PAIR, N_CLS, MAX_L), token_mask.dtype),
        ),
    )(so_cls_ids, enti_txt_embds, sW1, sb1, sW2, oW1, ob1, oW2,
      subj_ctx_embds, obj_ctx_embds, tm_sl)

    subj, obj = pl.pallas_call(
        _assemble_body,
        grid=(NCC, N_PAIR),
        in_specs=[
            pl.BlockSpec((C_BLK, D), lambda cc, p: (cc, 0)),
            pl.BlockSpec((C_BLK, SUF_L, D), lambda cc, p: (cc, 0, 0)),
            pl.BlockSpec((1, N_CTX, D), lambda cc, p: (p, 0, 0)),
            pl.BlockSpec((1, N_CTX, D), lambda cc, p: (p, 0, 0)),
        ],
        out_specs=[
            pl.BlockSpec((C_BLK, MAX_L, D), lambda cc, p: (p * NCC + cc, 0, 0)),
            pl.BlockSpec((C_BLK, MAX_L, D), lambda cc, p: (p * NCC + cc, 0, 0)),
        ],
        out_shape=(
            jax.ShapeDtypeStruct((N_PAIR * N_CLS, MAX_L, D), jnp.float32),
            jax.ShapeDtypeStruct((N_PAIR * N_CLS, MAX_L, D), jnp.float32),
        ),
    )(prefix_sl, suffix_sl, s_ctx, o_ctx)

    return subj, obj, tm3.reshape(N_PAIR * N_CLS, MAX_L)


# speedup vs baseline: 2.2591x; 2.2591x over previous
import jax
import jax.numpy as jnp
from jax.experimental import pallas as pl

N_PAIR = 32
N_CTX = 8
MAX_L = 40
D = 768
NUM_BASE = 92
NUM_NOVEL = 40
N_CLS = NUM_BASE + NUM_NOVEL
VOCAB = 1000
D_ENTI = 256
SUF_L = MAX_L - 1 - N_CTX

C_BLK = 22
NCC = N_CLS // C_BLK


def _ctx_body(ids_ref, enti_ref, sW1_ref, sb1_ref, sW2_ref, oW1_ref, ob1_ref,
              oW2_ref, sctx_in_ref, octx_in_ref, tm_ref,
              sctx_ref, octx_ref, tm3_ref):
    ids = ids_ref[:]
    iota = jax.lax.broadcasted_iota(jnp.int32, (N_PAIR, VOCAB), 1)
    s_oh = (ids[:, 0:1] == iota).astype(jnp.float32)
    o_oh = (ids[:, 1:2] == iota).astype(jnp.float32)
    enti = enti_ref[:]
    s_e = jnp.dot(s_oh, enti, preferred_element_type=jnp.float32)
    o_e = jnp.dot(o_oh, enti, preferred_element_type=jnp.float32)
    s_h = jnp.maximum(
        jnp.dot(s_e, sW1_ref[:], preferred_element_type=jnp.float32) + sb1_ref[:], 0.0)
    o_h = jnp.maximum(
        jnp.dot(o_e, oW1_ref[:], preferred_element_type=jnp.float32) + ob1_ref[:], 0.0)
    s_emb = jnp.dot(s_h, sW2_ref[:], preferred_element_type=jnp.float32)
    o_emb = jnp.dot(o_h, oW2_ref[:], preferred_element_type=jnp.float32)
    sctx_ref[:] = sctx_in_ref[:][None, :, :] + s_emb[:, None, :]
    octx_ref[:] = octx_in_ref[:][None, :, :] + o_emb[:, None, :]
    tm3_ref[:] = jnp.broadcast_to(tm_ref[:][None, :, :], (N_PAIR, N_CLS, MAX_L))


def _assemble_body(pre_ref, suf_ref, sctx_ref, octx_ref, subj_ref, obj_ref):
    pre = pre_ref[:]
    suf = suf_ref[:]
    s_ctx = jnp.broadcast_to(sctx_ref[0][None, :, :], (C_BLK, N_CTX, D))
    o_ctx = jnp.broadcast_to(octx_ref[0][None, :, :], (C_BLK, N_CTX, D))
    subj_ref[:] = jnp.concatenate([pre, s_ctx, suf], axis=1)
    obj_ref[:] = jnp.concatenate([pre, o_ctx, suf], axis=1)


def kernel(so_cls_ids, enti_txt_embds, prefix_embds, suffix_embds, token_mask,
           subj_ctx_embds, obj_ctx_embds, sW1, sb1, sW2, oW1, ob1, oW2):
    prefix_sl = prefix_embds[1:N_CLS + 1]
    suffix_sl = suffix_embds[1:N_CLS + 1]
    tm_sl = token_mask[1:N_CLS + 1]

    s_ctx, o_ctx, tm3 = pl.pallas_call(
        _ctx_body,
        out_shape=(
            jax.ShapeDtypeStruct((N_PAIR, N_CTX, D), jnp.float32),
            jax.ShapeDtypeStruct((N_PAIR, N_CTX, D), jnp.float32),
            jax.ShapeDtypeStruct((N_PAIR, N_CLS, MAX_L), token_mask.dtype),
        ),
    )(so_cls_ids, enti_txt_embds, sW1, sb1, sW2, oW1, ob1, oW2,
      subj_ctx_embds, obj_ctx_embds, tm_sl)

    subj, obj = pl.pallas_call(
        _assemble_body,
        grid=(NCC, N_PAIR),
        in_specs=[
            pl.BlockSpec((C_BLK, 1, D), lambda cc, p: (cc, 0, 0)),
            pl.BlockSpec((C_BLK, SUF_L, D), lambda cc, p: (cc, 0, 0)),
            pl.BlockSpec((1, N_CTX, D), lambda cc, p: (p, 0, 0)),
            pl.BlockSpec((1, N_CTX, D), lambda cc, p: (p, 0, 0)),
        ],
        out_specs=[
            pl.BlockSpec((C_BLK, MAX_L, D), lambda cc, p: (p * NCC + cc, 0, 0)),
            pl.BlockSpec((C_BLK, MAX_L, D), lambda cc, p: (p * NCC + cc, 0, 0)),
        ],
        out_shape=(
            jax.ShapeDtypeStruct((N_PAIR * N_CLS, MAX_L, D), jnp.float32),
            jax.ShapeDtypeStruct((N_PAIR * N_CLS, MAX_L, D), jnp.float32),
        ),
    )(prefix_sl, suffix_sl, s_ctx, o_ctx)

    return subj, obj, tm3.reshape(N_PAIR * N_CLS, MAX_L)


# speedup vs baseline: 2.3270x; 1.0301x over previous
import jax
import jax.numpy as jnp
from jax.experimental import pallas as pl

N_PAIR = 32
N_CTX = 8
MAX_L = 40
D = 768
NUM_BASE = 92
NUM_NOVEL = 40
N_CLS = NUM_BASE + NUM_NOVEL
VOCAB = 1000
D_ENTI = 256
SUF_L = MAX_L - 1 - N_CTX

C_BLK = 44
NCC = N_CLS // C_BLK


def _ctx_body(ids_ref, enti_ref, sW1_ref, sb1_ref, sW2_ref, oW1_ref, ob1_ref,
              oW2_ref, sctx_in_ref, octx_in_ref, tm_ref,
              sctx_ref, octx_ref, tm3_ref):
    ids = ids_ref[:]
    iota = jax.lax.broadcasted_iota(jnp.int32, (N_PAIR, VOCAB), 1)
    s_oh = (ids[:, 0:1] == iota).astype(jnp.float32)
    o_oh = (ids[:, 1:2] == iota).astype(jnp.float32)
    enti = enti_ref[:]
    s_e = jnp.dot(s_oh, enti, preferred_element_type=jnp.float32)
    o_e = jnp.dot(o_oh, enti, preferred_element_type=jnp.float32)
    s_h = jnp.maximum(
        jnp.dot(s_e, sW1_ref[:], preferred_element_type=jnp.float32) + sb1_ref[:], 0.0)
    o_h = jnp.maximum(
        jnp.dot(o_e, oW1_ref[:], preferred_element_type=jnp.float32) + ob1_ref[:], 0.0)
    s_emb = jnp.dot(s_h, sW2_ref[:], preferred_element_type=jnp.float32)
    o_emb = jnp.dot(o_h, oW2_ref[:], preferred_element_type=jnp.float32)
    sctx_ref[:] = sctx_in_ref[:][None, :, :] + s_emb[:, None, :]
    octx_ref[:] = octx_in_ref[:][None, :, :] + o_emb[:, None, :]
    tm3_ref[:] = jnp.broadcast_to(tm_ref[:][None, :, :], (N_PAIR, N_CLS, MAX_L))


def _assemble_body(pre_ref, suf_ref, sctx_ref, octx_ref, subj_ref, obj_ref):
    pre = pre_ref[:]
    suf = suf_ref[:]
    s_ctx = jnp.broadcast_to(sctx_ref[0][None, :, :], (C_BLK, N_CTX, D))
    o_ctx = jnp.broadcast_to(octx_ref[0][None, :, :], (C_BLK, N_CTX, D))
    subj_ref[:] = jnp.concatenate([pre, s_ctx, suf], axis=1)
    obj_ref[:] = jnp.concatenate([pre, o_ctx, suf], axis=1)


def kernel(so_cls_ids, enti_txt_embds, prefix_embds, suffix_embds, token_mask,
           subj_ctx_embds, obj_ctx_embds, sW1, sb1, sW2, oW1, ob1, oW2):
    prefix_sl = prefix_embds[1:N_CLS + 1]
    suffix_sl = suffix_embds[1:N_CLS + 1]
    tm_sl = token_mask[1:N_CLS + 1]

    s_ctx, o_ctx, tm3 = pl.pallas_call(
        _ctx_body,
        out_shape=(
            jax.ShapeDtypeStruct((N_PAIR, N_CTX, D), jnp.float32),
            jax.ShapeDtypeStruct((N_PAIR, N_CTX, D), jnp.float32),
            jax.ShapeDtypeStruct((N_PAIR, N_CLS, MAX_L), token_mask.dtype),
        ),
    )(so_cls_ids, enti_txt_embds, sW1, sb1, sW2, oW1, ob1, oW2,
      subj_ctx_embds, obj_ctx_embds, tm_sl)

    subj, obj = pl.pallas_call(
        _assemble_body,
        grid=(NCC, N_PAIR),
        in_specs=[
            pl.BlockSpec((C_BLK, 1, D), lambda cc, p: (cc, 0, 0)),
            pl.BlockSpec((C_BLK, SUF_L, D), lambda cc, p: (cc, 0, 0)),
            pl.BlockSpec((1, N_CTX, D), lambda cc, p: (p, 0, 0)),
            pl.BlockSpec((1, N_CTX, D), lambda cc, p: (p, 0, 0)),
        ],
        out_specs=[
            pl.BlockSpec((C_BLK, MAX_L, D), lambda cc, p: (p * NCC + cc, 0, 0)),
            pl.BlockSpec((C_BLK, MAX_L, D), lambda cc, p: (p * NCC + cc, 0, 0)),
        ],
        out_shape=(
            jax.ShapeDtypeStruct((N_PAIR * N_CLS, MAX_L, D), jnp.float32),
            jax.ShapeDtypeStruct((N_PAIR * N_CLS, MAX_L, D), jnp.float32),
        ),
    )(prefix_sl, suffix_sl, s_ctx, o_ctx)

    return subj, obj, tm3.reshape(N_PAIR * N_CLS, MAX_L)


# speedup vs baseline: 2.3316x; 1.0020x over previous
import jax
import jax.numpy as jnp
from jax.experimental import pallas as pl

N_PAIR = 32
N_CTX = 8
MAX_L = 40
D = 768
NUM_BASE = 92
NUM_NOVEL = 40
N_CLS = NUM_BASE + NUM_NOVEL
VOCAB = 1000
D_ENTI = 256
SUF_L = MAX_L - 1 - N_CTX

C_BLK = 66
NCC = N_CLS // C_BLK


def _ctx_body(ids_ref, enti_ref, sW1_ref, sb1_ref, sW2_ref, oW1_ref, ob1_ref,
              oW2_ref, sctx_in_ref, octx_in_ref, tm_ref,
              sctx_ref, octx_ref, tm3_ref):
    ids = ids_ref[:]
    iota = jax.lax.broadcasted_iota(jnp.int32, (N_PAIR, VOCAB), 1)
    s_oh = (ids[:, 0:1] == iota).astype(jnp.float32)
    o_oh = (ids[:, 1:2] == iota).astype(jnp.float32)
    enti = enti_ref[:]
    s_e = jnp.dot(s_oh, enti, preferred_element_type=jnp.float32)
    o_e = jnp.dot(o_oh, enti, preferred_element_type=jnp.float32)
    s_h = jnp.maximum(
        jnp.dot(s_e, sW1_ref[:], preferred_element_type=jnp.float32) + sb1_ref[:], 0.0)
    o_h = jnp.maximum(
        jnp.dot(o_e, oW1_ref[:], preferred_element_type=jnp.float32) + ob1_ref[:], 0.0)
    s_emb = jnp.dot(s_h, sW2_ref[:], preferred_element_type=jnp.float32)
    o_emb = jnp.dot(o_h, oW2_ref[:], preferred_element_type=jnp.float32)
    sctx_ref[:] = sctx_in_ref[:][None, :, :] + s_emb[:, None, :]
    octx_ref[:] = octx_in_ref[:][None, :, :] + o_emb[:, None, :]
    tm3_ref[:] = jnp.broadcast_to(tm_ref[:][None, :, :], (N_PAIR, N_CLS, MAX_L))


def _assemble_body(pre_ref, suf_ref, sctx_ref, octx_ref, subj_ref, obj_ref):
    pre = pre_ref[:]
    suf = suf_ref[:]
    s_ctx = jnp.broadcast_to(sctx_ref[0][None, :, :], (C_BLK, N_CTX, D))
    o_ctx = jnp.broadcast_to(octx_ref[0][None, :, :], (C_BLK, N_CTX, D))
    subj_ref[:] = jnp.concatenate([pre, s_ctx, suf], axis=1)
    obj_ref[:] = jnp.concatenate([pre, o_ctx, suf], axis=1)


def kernel(so_cls_ids, enti_txt_embds, prefix_embds, suffix_embds, token_mask,
           subj_ctx_embds, obj_ctx_embds, sW1, sb1, sW2, oW1, ob1, oW2):
    prefix_sl = prefix_embds[1:N_CLS + 1]
    suffix_sl = suffix_embds[1:N_CLS + 1]
    tm_sl = token_mask[1:N_CLS + 1]

    s_ctx, o_ctx, tm3 = pl.pallas_call(
        _ctx_body,
        out_shape=(
            jax.ShapeDtypeStruct((N_PAIR, N_CTX, D), jnp.float32),
            jax.ShapeDtypeStruct((N_PAIR, N_CTX, D), jnp.float32),
            jax.ShapeDtypeStruct((N_PAIR, N_CLS, MAX_L), token_mask.dtype),
        ),
    )(so_cls_ids, enti_txt_embds, sW1, sb1, sW2, oW1, ob1, oW2,
      subj_ctx_embds, obj_ctx_embds, tm_sl)

    subj, obj = pl.pallas_call(
        _assemble_body,
        grid=(NCC, N_PAIR),
        in_specs=[
            pl.BlockSpec((C_BLK, 1, D), lambda cc, p: (cc, 0, 0)),
            pl.BlockSpec((C_BLK, SUF_L, D), lambda cc, p: (cc, 0, 0)),
            pl.BlockSpec((1, N_CTX, D), lambda cc, p: (p, 0, 0)),
            pl.BlockSpec((1, N_CTX, D), lambda cc, p: (p, 0, 0)),
        ],
        out_specs=[
            pl.BlockSpec((C_BLK, MAX_L, D), lambda cc, p: (p * NCC + cc, 0, 0)),
            pl.BlockSpec((C_BLK, MAX_L, D), lambda cc, p: (p * NCC + cc, 0, 0)),
        ],
        out_shape=(
            jax.ShapeDtypeStruct((N_PAIR * N_CLS, MAX_L, D), jnp.float32),
            jax.ShapeDtypeStruct((N_PAIR * N_CLS, MAX_L, D), jnp.float32),
        ),
    )(prefix_sl, suffix_sl, s_ctx, o_ctx)

    return subj, obj, tm3.reshape(N_PAIR * N_CLS, MAX_L)
